# baseline (speedup 1.0000x reference)
import jax
import jax.numpy as jnp
from jax import lax
from jax.experimental import pallas as pl
from jax.experimental.pallas import tpu as pltpu
from jax.experimental.pallas import tpu_sc as plsc

N = 10000
E = 320000
MAX_NB = 32
D = 128
EDGE_INIT = 16
VOCAB = 512
DEPTH = 3

NW = 32
NODES_PER_W = 320
N_PAD = NW * NODES_PER_W
A_CHUNK = 2
A_ITERS = NODES_PER_W // A_CHUNK
E_PAD = 327680
NSEG = 2
SEG_E = E_PAD // NSEG
EW = SEG_E // NW
B_CHUNK = 80
B_ITERS = EW // B_CHUNK
NBUF = 4


def _wid():
    return lax.axis_index("s") * 2 + lax.axis_index("c")


def _tree_sum(vals):
    while len(vals) > 1:
        nxt = [vals[k] + vals[k + 1] for k in range(0, len(vals) - 1, 2)]
        if len(vals) % 2:
            nxt.append(vals[-1])
        vals = nxt
    return vals[0]


def _neisum_body(n2e_hbm, msg_hbm, out_hbm, idx_all, rows_v, out_all, *sems):
    wid = _wid()
    base_i = wid * (NODES_PER_W * MAX_NB)
    base_o = wid * NODES_PER_W
    CI = A_CHUNK * MAX_NB

    pltpu.sync_copy(n2e_hbm.at[pl.ds(base_i, NODES_PER_W * MAX_NB)], idx_all)

    def gather(buf, it):
        return pltpu.make_async_copy(
            msg_hbm.at[idx_all.at[pl.ds(it * CI, CI)]], rows_v.at[buf],
            sems[buf])

    for p in range(NBUF - 1):
        gather(p, p).start()

    def outer(o, carry):
        for b in range(NBUF):
            it = NBUF * o + b
            pb = (b + NBUF - 1) % NBUF

            @pl.when(it + NBUF - 1 < A_ITERS)
            def _():
                gather(pb, it + NBUF - 1).start()

            gather(b, it).wait()
            for i in range(A_CHUNK):
                for g in range(D // 16):
                    sl = pl.ds(g * 16, 16)
                    out_all[it * A_CHUNK + i, sl] = _tree_sum(
                        [rows_v[b, i * MAX_NB + j, sl] for j in range(MAX_NB)])
        return carry

    lax.fori_loop(0, A_ITERS // NBUF, outer, 0)
    pltpu.sync_copy(out_all, out_hbm.at[pl.ds(base_o, NODES_PER_W)])


def _neisum(n2e_flat, msg):
    mesh = plsc.VectorSubcoreMesh(core_axis_name="c", subcore_axis_name="s",
                                  num_cores=2, num_subcores=16)
    f = pl.kernel(
        _neisum_body,
        out_type=jax.ShapeDtypeStruct((N_PAD, D), jnp.float32),
        mesh=mesh,
        scratch_types=[
            pltpu.VMEM((NODES_PER_W * MAX_NB,), jnp.int32),
            pltpu.VMEM((NBUF, A_CHUNK * MAX_NB, D), jnp.float32),
            pltpu.VMEM((NODES_PER_W, D), jnp.float32),
        ] + [pltpu.SemaphoreType.DMA] * NBUF,
    )
    return f(n2e_flat, msg)


def _make_edgemix_body(seg):
    seg_base = seg * SEG_E

    def body(e2n_hbm, brev_hbm, amsg_hbm, msg_hbm, out_hbm,
             idxa_all, idxr_all, arows_v, rrows_v, *sems):
        semas = sems[:NBUF]
        semrs = sems[NBUF:2 * NBUF]
        semos = sems[2 * NBUF:]
        base = seg_base + _wid() * EW

        pltpu.sync_copy(e2n_hbm.at[pl.ds(base, EW)], idxa_all)
        pltpu.sync_copy(brev_hbm.at[pl.ds(base, EW)], idxr_all)

        def ga(buf, it):
            return pltpu.make_async_copy(
                amsg_hbm.at[idxa_all.at[pl.ds(it * B_CHUNK, B_CHUNK)]],
                arows_v.at[buf], semas[buf])

        def gr(buf, it):
            return pltpu.make_async_copy(
                msg_hbm.at[idxr_all.at[pl.ds(it * B_CHUNK, B_CHUNK)]],
                rrows_v.at[buf], semrs[buf])

        def wr(buf, it):
            return pltpu.make_async_copy(
                arows_v.at[buf],
                out_hbm.at[pl.ds(base - seg_base + it * B_CHUNK, B_CHUNK)],
                semos[buf])

        for p in range(NBUF - 1):
            ga(p, p).start()
            gr(p, p).start()

        def outer(o, carry):
            for b in range(NBUF):
                it = NBUF * o + b
                pb = (b + NBUF - 1) % NBUF

                @pl.when(it + NBUF - 1 < B_ITERS)
                def _():
                    @pl.when(it >= 1)
                    def _():
                        wr(pb, it - 1).wait()
                    ga(pb, it + NBUF - 1).start()
                    gr(pb, it + NBUF - 1).start()

                ga(b, it).wait()
                gr(b, it).wait()

                def row(r, c2):
                    for g in range(D // 16):
                        sl = pl.ds(g * 16, 16)
                        arows_v[b, r, sl] = (arows_v[b, r, sl]
                                             - rrows_v[b, r, sl])
                    return c2

                lax.fori_loop(0, B_CHUNK, row, 0)
                wr(b, it).start()
            return carry

        lax.fori_loop(0, B_ITERS // NBUF, outer, 0)
        for b in range(NBUF):
            wr(b, B_ITERS - NBUF + b).wait()

    return body


def _edgemix(seg, e2n_pad, brev_pad, amsg, msg):
    mesh = plsc.VectorSubcoreMesh(core_axis_name="c", subcore_axis_name="s",
                                  num_cores=2, num_subcores=16)
    f = pl.kernel(
        _make_edgemix_body(seg),
        out_type=jax.ShapeDtypeStruct((SEG_E, D), jnp.float32),
        mesh=mesh,
        scratch_types=[
            pltpu.VMEM((EW,), jnp.int32),
            pltpu.VMEM((EW,), jnp.int32),
            pltpu.VMEM((NBUF, B_CHUNK, D), jnp.float32),
            pltpu.VMEM((NBUF, B_CHUNK, D), jnp.float32),
        ] + [pltpu.SemaphoreType.DMA] * (3 * NBUF),
    )
    return f(e2n_pad, brev_pad, amsg, msg)


_FE_TILE = 2560


def _fe_body(fe_ref, weT_ref, out_ref):
    i = pl.program_id(0)
    x = jnp.dot(fe_ref[...], weT_ref[...], preferred_element_type=jnp.float32)
    grow = i * _FE_TILE + lax.broadcasted_iota(jnp.int32, (_FE_TILE, D), 0)
    out_ref[...] = jnp.where(grow == 0, 0.0, x)


def _fe(f_pad, w_edgeT):
    return pl.pallas_call(
        _fe_body,
        grid=(E_PAD // _FE_TILE,),
        in_specs=[
            pl.BlockSpec((_FE_TILE, EDGE_INIT), lambda i: (i, 0)),
            pl.BlockSpec((EDGE_INIT, D), lambda i: (0, 0)),
        ],
        out_specs=pl.BlockSpec((_FE_TILE, D), lambda i: (i, 0)),
        out_shape=jax.ShapeDtypeStruct((E_PAD, D), jnp.float32),
    )(f_pad, w_edgeT)


_G_TILE = 2048


def _make_gru_body(seg):
    tile0 = seg * (SEG_E // _G_TILE)

    def body(fe_ref, mp_ref, weT_ref, wihT_ref, whhT_ref, bih_ref, bhh_ref,
             io_ref, out_ref):
        i = pl.program_id(0)
        wcombT = jnp.dot(weT_ref[...], wihT_ref[...],
                         preferred_element_type=jnp.float32)
        gi = jnp.dot(fe_ref[...], wcombT,
                     preferred_element_type=jnp.float32) + bih_ref[...]
        h = mp_ref[...]
        gh = jnp.dot(h, whhT_ref[...],
                     preferred_element_type=jnp.float32) + bhh_ref[...]
        r = jax.nn.sigmoid(gi[:, :D] + gh[:, :D])
        z = jax.nn.sigmoid(gi[:, D:2 * D] + gh[:, D:2 * D])
        n = jnp.tanh(gi[:, 2 * D:] + r * gh[:, 2 * D:])
        out = (1.0 - z) * n + z * h
        grow = (tile0 + i) * _G_TILE + lax.broadcasted_iota(
            jnp.int32, (_G_TILE, D), 0)
        out_ref[...] = jnp.where(grow == 0, 0.0, out)

    return body


def _gru_seg(seg, f_pad, mp, w_edgeT, w_ihT, w_hhT, bih2, bhh2, msg_io):
    tile0 = seg * (SEG_E // _G_TILE)
    return pl.pallas_call(
        _make_gru_body(seg),
        grid=(SEG_E // _G_TILE,),
        in_specs=[
            pl.BlockSpec((_G_TILE, EDGE_INIT), lambda i: (i + tile0, 0)),
            pl.BlockSpec((_G_TILE, D), lambda i: (i, 0)),
            pl.BlockSpec((EDGE_INIT, D), lambda i: (0, 0)),
            pl.BlockSpec((D, 3 * D), lambda i: (0, 0)),
            pl.BlockSpec((D, 3 * D), lambda i: (0, 0)),
            pl.BlockSpec((1, 3 * D), lambda i: (0, 0)),
            pl.BlockSpec((1, 3 * D), lambda i: (0, 0)),
            pl.BlockSpec(memory_space=pl.ANY),
        ],
        out_specs=pl.BlockSpec((_G_TILE, D), lambda i: (i + tile0, 0)),
        out_shape=jax.ShapeDtypeStruct((E_PAD, D), jnp.float32),
        input_output_aliases={7: 0},
    )(f_pad, mp, w_edgeT, w_ihT, w_hhT, bih2, bhh2, msg_io)


_R_TILE = 1000


def _ro_body(fn_ref, am_ref, emb_ref, wroT_ref, bro_ref, out_ref):
    ids = lax.broadcasted_iota(jnp.int32, (_R_TILE, VOCAB), 1).astype(jnp.float32)
    oh = (fn_ref[...] == ids).astype(jnp.float32)
    nf = jnp.dot(oh, emb_ref[...], preferred_element_type=jnp.float32)
    w = wroT_ref[...]
    t = (jnp.dot(nf, w[:D, :], preferred_element_type=jnp.float32)
         + jnp.dot(am_ref[...], w[D:, :], preferred_element_type=jnp.float32))
    out_ref[...] = jnp.maximum(t + bro_ref[...], 0.0)


def _ro(fn_f32, amsg, emb, w_roT, bro2):
    return pl.pallas_call(
        _ro_body,
        grid=(N // _R_TILE,),
        in_specs=[
            pl.BlockSpec((_R_TILE, 1), lambda i: (i, 0)),
            pl.BlockSpec((_R_TILE, D), lambda i: (i, 0)),
            pl.BlockSpec((VOCAB, D), lambda i: (0, 0)),
            pl.BlockSpec((2 * D, D), lambda i: (0, 0)),
            pl.BlockSpec((1, D), lambda i: (0, 0)),
        ],
        out_specs=pl.BlockSpec((_R_TILE, D), lambda i: (i, 0)),
        out_shape=jax.ShapeDtypeStruct((N, D), jnp.float32),
    )(fn_f32, amsg, emb, w_roT, bro2)


def kernel(f_nodes, f_edges, node2edge, edge2node, b2revb, emb, w_edge,
           w_ih, w_hh, b_ih, b_hh, w_ro, b_ro):
    w_edgeT = w_edge.T
    w_ihT = w_ih.T
    w_hhT = w_hh.T
    w_roT = w_ro.T
    bih2 = b_ih.reshape(1, 3 * D)
    bhh2 = b_hh.reshape(1, 3 * D)
    bro2 = b_ro.reshape(1, D)
    f_pad = jnp.pad(f_edges, ((0, E_PAD - E), (0, 0)))
    n2e_flat = jnp.pad(node2edge.reshape(-1).astype(jnp.int32),
                       (0, (N_PAD - N) * MAX_NB))
    e2n_pad = jnp.pad(edge2node.astype(jnp.int32), (0, E_PAD - E))
    brev_pad = jnp.pad(b2revb.astype(jnp.int32), (0, E_PAD - E))
    fn_f32 = f_nodes.astype(jnp.float32).reshape(N, 1)

    msg = _fe(f_pad, w_edgeT)
    alias_src = jnp.zeros((E_PAD, D), jnp.float32)
    for _ in range(DEPTH - 1):
        amsg = _neisum(n2e_flat, msg)
        mps = [_edgemix(s, e2n_pad, brev_pad, amsg, msg) for s in range(NSEG)]
        io = alias_src
        for s in range(NSEG):
            io = _gru_seg(s, f_pad, mps[s], w_edgeT, w_ihT, w_hhT,
                          bih2, bhh2, io)
        alias_src = msg
        msg = io
    amsg = _neisum(n2e_flat, msg)
    return _ro(fn_f32, amsg, emb, w_roT, bro2)


# speedup vs baseline: 1.0180x; 1.0180x over previous
import jax
import jax.numpy as jnp
from jax import lax
from jax.experimental import pallas as pl
from jax.experimental.pallas import tpu as pltpu
from jax.experimental.pallas import tpu_sc as plsc

N = 10000
E = 320000
MAX_NB = 32
D = 128
EDGE_INIT = 16
VOCAB = 512
DEPTH = 3

NW = 32
NODES_PER_W = 320
N_PAD = NW * NODES_PER_W
A_CHUNK = 2
A_ITERS = NODES_PER_W // A_CHUNK
E_PAD = 327680
NSEG = 2
SEG_E = E_PAD // NSEG
EW = SEG_E // NW
B_CHUNK = 80
B_ITERS = EW // B_CHUNK
NBUF = 4


def _wid():
    return lax.axis_index("s") * 2 + lax.axis_index("c")


def _tree_sum(vals):
    while len(vals) > 1:
        nxt = [vals[k] + vals[k + 1] for k in range(0, len(vals) - 1, 2)]
        if len(vals) % 2:
            nxt.append(vals[-1])
        vals = nxt
    return vals[0]


def _neisum_body(n2e_hbm, msg_hbm, out_hbm, idx_all, rows_v, acc_v, *sems):
    semg = sems[:NBUF]
    semo = sems[NBUF:]
    wid = _wid()
    base_i = wid * (NODES_PER_W * MAX_NB)
    base_o = wid * NODES_PER_W
    CI = A_CHUNK * MAX_NB

    pltpu.sync_copy(n2e_hbm.at[pl.ds(base_i, NODES_PER_W * MAX_NB)], idx_all)

    def gather(buf, it):
        return pltpu.make_async_copy(
            msg_hbm.at[idx_all.at[pl.ds(it * CI, CI)]], rows_v.at[buf],
            semg[buf])

    def wrn(buf, it):
        return pltpu.make_async_copy(
            acc_v.at[buf],
            out_hbm.at[pl.ds(base_o + it * A_CHUNK, A_CHUNK)], semo[buf])

    for p in range(NBUF - 1):
        gather(p, p).start()

    def outer(o, carry):
        for b in range(NBUF):
            it = NBUF * o + b
            pb = (b + NBUF - 1) % NBUF

            @pl.when(it + NBUF - 1 < A_ITERS)
            def _():
                gather(pb, it + NBUF - 1).start()

            gather(b, it).wait()

            @pl.when(it >= NBUF)
            def _():
                wrn(b, it - NBUF).wait()

            for i in range(A_CHUNK):
                for g in range(D // 16):
                    sl = pl.ds(g * 16, 16)
                    acc_v[b, i, sl] = _tree_sum(
                        [rows_v[b, i * MAX_NB + j, sl] for j in range(MAX_NB)])
            wrn(b, it).start()
        return carry

    lax.fori_loop(0, A_ITERS // NBUF, outer, 0)
    for b in range(NBUF):
        wrn(b, A_ITERS - NBUF + b).wait()


def _neisum(n2e_flat, msg):
    mesh = plsc.VectorSubcoreMesh(core_axis_name="c", subcore_axis_name="s",
                                  num_cores=2, num_subcores=16)
    f = pl.kernel(
        _neisum_body,
        out_type=jax.ShapeDtypeStruct((N_PAD, D), jnp.float32),
        mesh=mesh,
        scratch_types=[
            pltpu.VMEM((NODES_PER_W * MAX_NB,), jnp.int32),
            pltpu.VMEM((NBUF, A_CHUNK * MAX_NB, D), jnp.float32),
            pltpu.VMEM((NBUF, A_CHUNK, D), jnp.float32),
        ] + [pltpu.SemaphoreType.DMA] * (2 * NBUF),
    )
    return f(n2e_flat, msg)


def _make_edgemix_body(seg):
    seg_base = seg * SEG_E

    def body(e2n_hbm, brev_hbm, amsg_hbm, msg_hbm, out_hbm,
             idxa_all, idxr_all, arows_v, rrows_v, *sems):
        semas = sems[:NBUF]
        semrs = sems[NBUF:2 * NBUF]
        semos = sems[2 * NBUF:]
        base = seg_base + _wid() * EW

        pltpu.sync_copy(e2n_hbm.at[pl.ds(base, EW)], idxa_all)
        pltpu.sync_copy(brev_hbm.at[pl.ds(base, EW)], idxr_all)

        def ga(buf, it):
            return pltpu.make_async_copy(
                amsg_hbm.at[idxa_all.at[pl.ds(it * B_CHUNK, B_CHUNK)]],
                arows_v.at[buf], semas[buf])

        def gr(buf, it):
            return pltpu.make_async_copy(
                msg_hbm.at[idxr_all.at[pl.ds(it * B_CHUNK, B_CHUNK)]],
                rrows_v.at[buf], semrs[buf])

        def wr(buf, it):
            return pltpu.make_async_copy(
                arows_v.at[buf],
                out_hbm.at[pl.ds(base - seg_base + it * B_CHUNK, B_CHUNK)],
                semos[buf])

        for p in range(NBUF - 1):
            ga(p, p).start()
            gr(p, p).start()

        def outer(o, carry):
            for b in range(NBUF):
                it = NBUF * o + b
                pb = (b + NBUF - 1) % NBUF

                @pl.when(it + NBUF - 1 < B_ITERS)
                def _():
                    @pl.when(it >= 1)
                    def _():
                        wr(pb, it - 1).wait()
                    ga(pb, it + NBUF - 1).start()
                    gr(pb, it + NBUF - 1).start()

                ga(b, it).wait()
                gr(b, it).wait()

                def row(r, c2):
                    for g in range(D // 16):
                        sl = pl.ds(g * 16, 16)
                        arows_v[b, r, sl] = (arows_v[b, r, sl]
                                             - rrows_v[b, r, sl])
                    return c2

                lax.fori_loop(0, B_CHUNK, row, 0)
                wr(b, it).start()
            return carry

        lax.fori_loop(0, B_ITERS // NBUF, outer, 0)
        for b in range(NBUF):
            wr(b, B_ITERS - NBUF + b).wait()

    return body


def _edgemix(seg, e2n_pad, brev_pad, amsg, msg):
    mesh = plsc.VectorSubcoreMesh(core_axis_name="c", subcore_axis_name="s",
                                  num_cores=2, num_subcores=16)
    f = pl.kernel(
        _make_edgemix_body(seg),
        out_type=jax.ShapeDtypeStruct((SEG_E, D), jnp.float32),
        mesh=mesh,
        scratch_types=[
            pltpu.VMEM((EW,), jnp.int32),
            pltpu.VMEM((EW,), jnp.int32),
            pltpu.VMEM((NBUF, B_CHUNK, D), jnp.float32),
            pltpu.VMEM((NBUF, B_CHUNK, D), jnp.float32),
        ] + [pltpu.SemaphoreType.DMA] * (3 * NBUF),
    )
    return f(e2n_pad, brev_pad, amsg, msg)


_FE_TILE = 2560


def _fe_body(fe_ref, weT_ref, out_ref):
    i = pl.program_id(0)
    x = jnp.dot(fe_ref[...], weT_ref[...], preferred_element_type=jnp.float32)
    grow = i * _FE_TILE + lax.broadcasted_iota(jnp.int32, (_FE_TILE, D), 0)
    out_ref[...] = jnp.where(grow == 0, 0.0, x)


def _fe(f_pad, w_edgeT):
    return pl.pallas_call(
        _fe_body,
        grid=(E_PAD // _FE_TILE,),
        in_specs=[
            pl.BlockSpec((_FE_TILE, EDGE_INIT), lambda i: (i, 0)),
            pl.BlockSpec((EDGE_INIT, D), lambda i: (0, 0)),
        ],
        out_specs=pl.BlockSpec((_FE_TILE, D), lambda i: (i, 0)),
        out_shape=jax.ShapeDtypeStruct((E_PAD, D), jnp.float32),
    )(f_pad, w_edgeT)


_G_TILE = 2048


def _make_gru_body(seg):
    tile0 = seg * (SEG_E // _G_TILE)

    def body(fe_ref, mp_ref, weT_ref, wihT_ref, whhT_ref, bih_ref, bhh_ref,
             io_ref, out_ref):
        i = pl.program_id(0)
        wcombT = jnp.dot(weT_ref[...], wihT_ref[...],
                         preferred_element_type=jnp.float32)
        gi = jnp.dot(fe_ref[...], wcombT,
                     preferred_element_type=jnp.float32) + bih_ref[...]
        h = mp_ref[...]
        gh = jnp.dot(h, whhT_ref[...],
                     preferred_element_type=jnp.float32) + bhh_ref[...]
        r = jax.nn.sigmoid(gi[:, :D] + gh[:, :D])
        z = jax.nn.sigmoid(gi[:, D:2 * D] + gh[:, D:2 * D])
        n = jnp.tanh(gi[:, 2 * D:] + r * gh[:, 2 * D:])
        out = (1.0 - z) * n + z * h
        grow = (tile0 + i) * _G_TILE + lax.broadcasted_iota(
            jnp.int32, (_G_TILE, D), 0)
        out_ref[...] = jnp.where(grow == 0, 0.0, out)

    return body


def _gru_seg(seg, f_pad, mp, w_edgeT, w_ihT, w_hhT, bih2, bhh2, msg_io):
    tile0 = seg * (SEG_E // _G_TILE)
    return pl.pallas_call(
        _make_gru_body(seg),
        grid=(SEG_E // _G_TILE,),
        in_specs=[
            pl.BlockSpec((_G_TILE, EDGE_INIT), lambda i: (i + tile0, 0)),
            pl.BlockSpec((_G_TILE, D), lambda i: (i, 0)),
            pl.BlockSpec((EDGE_INIT, D), lambda i: (0, 0)),
            pl.BlockSpec((D, 3 * D), lambda i: (0, 0)),
            pl.BlockSpec((D, 3 * D), lambda i: (0, 0)),
            pl.BlockSpec((1, 3 * D), lambda i: (0, 0)),
            pl.BlockSpec((1, 3 * D), lambda i: (0, 0)),
            pl.BlockSpec(memory_space=pl.ANY),
        ],
        out_specs=pl.BlockSpec((_G_TILE, D), lambda i: (i + tile0, 0)),
        out_shape=jax.ShapeDtypeStruct((E_PAD, D), jnp.float32),
        input_output_aliases={7: 0},
    )(f_pad, mp, w_edgeT, w_ihT, w_hhT, bih2, bhh2, msg_io)


_R_TILE = 1000


def _ro_body(fn_ref, am_ref, emb_ref, wroT_ref, bro_ref, out_ref):
    ids = lax.broadcasted_iota(jnp.int32, (_R_TILE, VOCAB), 1).astype(jnp.float32)
    oh = (fn_ref[...] == ids).astype(jnp.float32)
    nf = jnp.dot(oh, emb_ref[...], preferred_element_type=jnp.float32)
    w = wroT_ref[...]
    t = (jnp.dot(nf, w[:D, :], preferred_element_type=jnp.float32)
         + jnp.dot(am_ref[...], w[D:, :], preferred_element_type=jnp.float32))
    out_ref[...] = jnp.maximum(t + bro_ref[...], 0.0)


def _ro(fn_f32, amsg, emb, w_roT, bro2):
    return pl.pallas_call(
        _ro_body,
        grid=(N // _R_TILE,),
        in_specs=[
            pl.BlockSpec((_R_TILE, 1), lambda i: (i, 0)),
            pl.BlockSpec((_R_TILE, D), lambda i: (i, 0)),
            pl.BlockSpec((VOCAB, D), lambda i: (0, 0)),
            pl.BlockSpec((2 * D, D), lambda i: (0, 0)),
            pl.BlockSpec((1, D), lambda i: (0, 0)),
        ],
        out_specs=pl.BlockSpec((_R_TILE, D), lambda i: (i, 0)),
        out_shape=jax.ShapeDtypeStruct((N, D), jnp.float32),
    )(fn_f32, amsg, emb, w_roT, bro2)


def kernel(f_nodes, f_edges, node2edge, edge2node, b2revb, emb, w_edge,
           w_ih, w_hh, b_ih, b_hh, w_ro, b_ro):
    w_edgeT = w_edge.T
    w_ihT = w_ih.T
    w_hhT = w_hh.T
    w_roT = w_ro.T
    bih2 = b_ih.reshape(1, 3 * D)
    bhh2 = b_hh.reshape(1, 3 * D)
    bro2 = b_ro.reshape(1, D)
    f_pad = jnp.pad(f_edges, ((0, E_PAD - E), (0, 0)))
    n2e_flat = jnp.pad(node2edge.reshape(-1).astype(jnp.int32),
                       (0, (N_PAD - N) * MAX_NB))
    e2n_pad = jnp.pad(edge2node.astype(jnp.int32), (0, E_PAD - E))
    brev_pad = jnp.pad(b2revb.astype(jnp.int32), (0, E_PAD - E))
    fn_f32 = f_nodes.astype(jnp.float32).reshape(N, 1)

    msg = _fe(f_pad, w_edgeT)
    for _ in range(DEPTH - 1):
        amsg = _neisum(n2e_flat, msg)
        mps = [_edgemix(s, e2n_pad, brev_pad, amsg, msg) for s in range(NSEG)]
        io = msg
        for s in range(NSEG):
            io = _gru_seg(s, f_pad, mps[s], w_edgeT, w_ihT, w_hhT,
                          bih2, bhh2, io)
        msg = io
    amsg = _neisum(n2e_flat, msg)
    return _ro(fn_f32, amsg, emb, w_roT, bro2)


# speedup vs baseline: 1.0563x; 1.0377x over previous
import jax
import jax.numpy as jnp
from jax import lax
from jax.experimental import pallas as pl
from jax.experimental.pallas import tpu as pltpu
from jax.experimental.pallas import tpu_sc as plsc

N = 10000
E = 320000
MAX_NB = 32
D = 128
EDGE_INIT = 16
VOCAB = 512
DEPTH = 3

NW = 32
N_PAD = 10240
E_PAD = 327680
A_CHUNK = 2
B_CHUNK = 80
NBUF = 4

NA, NB = 440, 200
EA, EB = 14080, 6400
NMX = max(NA, NB)
EMX = max(EA, EB)
N_IDX_PAD = N_PAD + NMX
E_IDX_PAD = E_PAD + EMX


def _tree_sum(vals):
    while len(vals) > 1:
        nxt = [vals[k] + vals[k + 1] for k in range(0, len(vals) - 1, 2)]
        if len(vals) % 2:
            nxt.append(vals[-1])
        vals = nxt
    return vals[0]


def _neisum_body(n2e_hbm, msg_hbm, out_hbm, idx_all, rows_v, out_all, *sems):
    c = lax.axis_index("c")
    s = lax.axis_index("s")
    base_o = jnp.where(c == 0, s * NA, 16 * NA + s * NB)
    n_iters = jnp.where(c == 0, NA // A_CHUNK, NB // A_CHUNK)
    CI = A_CHUNK * MAX_NB

    pltpu.sync_copy(n2e_hbm.at[pl.ds(base_o * MAX_NB, NMX * MAX_NB)], idx_all)

    def gather(buf, it):
        return pltpu.make_async_copy(
            msg_hbm.at[idx_all.at[pl.ds(it * CI, CI)]], rows_v.at[buf],
            sems[buf])

    for p in range(NBUF - 1):
        gather(p, p).start()

    def outer(o, carry):
        for b in range(NBUF):
            it = NBUF * o + b
            pb = (b + NBUF - 1) % NBUF

            @pl.when(it + NBUF - 1 < n_iters)
            def _():
                gather(pb, it + NBUF - 1).start()

            gather(b, it).wait()
            for i in range(A_CHUNK):
                for g in range(D // 16):
                    sl = pl.ds(g * 16, 16)
                    out_all[it * A_CHUNK + i, sl] = _tree_sum(
                        [rows_v[b, i * MAX_NB + j, sl] for j in range(MAX_NB)])
        return carry

    lax.fori_loop(0, n_iters // NBUF, outer, 0)

    @pl.when(c == 0)
    def _():
        pltpu.sync_copy(out_all.at[pl.ds(0, NA)],
                        out_hbm.at[pl.ds(s * NA, NA)])

    @pl.when(c == 1)
    def _():
        pltpu.sync_copy(out_all.at[pl.ds(0, NB)],
                        out_hbm.at[pl.ds(16 * NA + s * NB, NB)])


def _neisum(n2e_flat, msg):
    mesh = plsc.VectorSubcoreMesh(core_axis_name="c", subcore_axis_name="s",
                                  num_cores=2, num_subcores=16)
    f = pl.kernel(
        _neisum_body,
        out_type=jax.ShapeDtypeStruct((N_PAD, D), jnp.float32),
        mesh=mesh,
        scratch_types=[
            pltpu.VMEM((NMX * MAX_NB,), jnp.int32),
            pltpu.VMEM((NBUF, A_CHUNK * MAX_NB, D), jnp.float32),
            pltpu.VMEM((NMX, D), jnp.float32),
        ] + [pltpu.SemaphoreType.DMA] * NBUF,
    )
    return f(n2e_flat, msg)


def _edgemix_body(e2n_hbm, brev_hbm, amsg_hbm, msg_hbm, out_hbm,
                  idxa_all, idxr_all, arows_v, rrows_v, *sems):
    semas = sems[:NBUF]
    semrs = sems[NBUF:2 * NBUF]
    semos = sems[2 * NBUF:]
    c = lax.axis_index("c")
    s = lax.axis_index("s")
    base = jnp.where(c == 0, s * EA, 16 * EA + s * EB)
    n_iters = jnp.where(c == 0, EA // B_CHUNK, EB // B_CHUNK)

    pltpu.sync_copy(e2n_hbm.at[pl.ds(base, EMX)], idxa_all)
    pltpu.sync_copy(brev_hbm.at[pl.ds(base, EMX)], idxr_all)

    def ga(buf, it):
        return pltpu.make_async_copy(
            amsg_hbm.at[idxa_all.at[pl.ds(it * B_CHUNK, B_CHUNK)]],
            arows_v.at[buf], semas[buf])

    def gr(buf, it):
        return pltpu.make_async_copy(
            msg_hbm.at[idxr_all.at[pl.ds(it * B_CHUNK, B_CHUNK)]],
            rrows_v.at[buf], semrs[buf])

    def wr(buf, it):
        return pltpu.make_async_copy(
            arows_v.at[buf], out_hbm.at[pl.ds(base + it * B_CHUNK, B_CHUNK)],
            semos[buf])

    for p in range(NBUF - 1):
        ga(p, p).start()
        gr(p, p).start()

    def outer(o, carry):
        for b in range(NBUF):
            it = NBUF * o + b
            pb = (b + NBUF - 1) % NBUF

            @pl.when(it + NBUF - 1 < n_iters)
            def _():
                @pl.when(it >= 1)
                def _():
                    wr(pb, it - 1).wait()
                ga(pb, it + NBUF - 1).start()
                gr(pb, it + NBUF - 1).start()

            ga(b, it).wait()
            gr(b, it).wait()

            def row(r, c2):
                for g in range(D // 16):
                    sl = pl.ds(g * 16, 16)
                    arows_v[b, r, sl] = arows_v[b, r, sl] - rrows_v[b, r, sl]
                return c2

            lax.fori_loop(0, B_CHUNK, row, 0)
            wr(b, it).start()
        return carry

    lax.fori_loop(0, n_iters // NBUF, outer, 0)
    for b in range(NBUF):
        wr(b, n_iters - NBUF + b).wait()


def _edgemix(e2n_pad, brev_pad, amsg, msg):
    mesh = plsc.VectorSubcoreMesh(core_axis_name="c", subcore_axis_name="s",
                                  num_cores=2, num_subcores=16)
    f = pl.kernel(
        _edgemix_body,
        out_type=jax.ShapeDtypeStruct((E_PAD, D), jnp.float32),
        mesh=mesh,
        scratch_types=[
            pltpu.VMEM((EMX,), jnp.int32),
            pltpu.VMEM((EMX,), jnp.int32),
            pltpu.VMEM((NBUF, B_CHUNK, D), jnp.float32),
            pltpu.VMEM((NBUF, B_CHUNK, D), jnp.float32),
        ] + [pltpu.SemaphoreType.DMA] * (3 * NBUF),
    )
    return f(e2n_pad, brev_pad, amsg, msg)


_FE_TILE = 2560


def _fe_body(fe_ref, weT_ref, out_ref):
    i = pl.program_id(0)
    x = jnp.dot(fe_ref[...], weT_ref[...], preferred_element_type=jnp.float32)
    grow = i * _FE_TILE + lax.broadcasted_iota(jnp.int32, (_FE_TILE, D), 0)
    out_ref[...] = jnp.where(grow == 0, 0.0, x)


def _fe(f_edges, w_edgeT):
    return pl.pallas_call(
        _fe_body,
        grid=(E // _FE_TILE,),
        in_specs=[
            pl.BlockSpec((_FE_TILE, EDGE_INIT), lambda i: (i, 0)),
            pl.BlockSpec((EDGE_INIT, D), lambda i: (0, 0)),
        ],
        out_specs=pl.BlockSpec((_FE_TILE, D), lambda i: (i, 0)),
        out_shape=jax.ShapeDtypeStruct((E, D), jnp.float32),
    )(f_edges, w_edgeT)


_G_TILE = 2000


def _gru_body(fe_ref, mp_ref, weT_ref, wihT_ref, whhT_ref, bih_ref, bhh_ref,
              out_ref):
    i = pl.program_id(0)
    wcombT = jnp.dot(weT_ref[...], wihT_ref[...],
                     preferred_element_type=jnp.float32)
    gi = jnp.dot(fe_ref[...], wcombT,
                 preferred_element_type=jnp.float32) + bih_ref[...]
    h = mp_ref[...]
    gh = jnp.dot(h, whhT_ref[...],
                 preferred_element_type=jnp.float32) + bhh_ref[...]
    r = jax.nn.sigmoid(gi[:, :D] + gh[:, :D])
    z = jax.nn.sigmoid(gi[:, D:2 * D] + gh[:, D:2 * D])
    n = jnp.tanh(gi[:, 2 * D:] + r * gh[:, 2 * D:])
    out = (1.0 - z) * n + z * h
    grow = i * _G_TILE + lax.broadcasted_iota(jnp.int32, (_G_TILE, D), 0)
    out_ref[...] = jnp.where(grow == 0, 0.0, out)


def _gru(f_edges, mp, w_edgeT, w_ihT, w_hhT, bih2, bhh2):
    return pl.pallas_call(
        _gru_body,
        grid=(E // _G_TILE,),
        in_specs=[
            pl.BlockSpec((_G_TILE, EDGE_INIT), lambda i: (i, 0)),
            pl.BlockSpec((_G_TILE, D), lambda i: (i, 0)),
            pl.BlockSpec((EDGE_INIT, D), lambda i: (0, 0)),
            pl.BlockSpec((D, 3 * D), lambda i: (0, 0)),
            pl.BlockSpec((D, 3 * D), lambda i: (0, 0)),
            pl.BlockSpec((1, 3 * D), lambda i: (0, 0)),
            pl.BlockSpec((1, 3 * D), lambda i: (0, 0)),
        ],
        out_specs=pl.BlockSpec((_G_TILE, D), lambda i: (i, 0)),
        out_shape=jax.ShapeDtypeStruct((E, D), jnp.float32),
    )(f_edges, mp, w_edgeT, w_ihT, w_hhT, bih2, bhh2)


_R_TILE = 1000


def _ro_body(fn_ref, am_ref, emb_ref, wroT_ref, bro_ref, out_ref):
    ids = lax.broadcasted_iota(jnp.int32, (_R_TILE, VOCAB), 1).astype(jnp.float32)
    oh = (fn_ref[...] == ids).astype(jnp.float32)
    nf = jnp.dot(oh, emb_ref[...], preferred_element_type=jnp.float32)
    w = wroT_ref[...]
    t = (jnp.dot(nf, w[:D, :], preferred_element_type=jnp.float32)
         + jnp.dot(am_ref[...], w[D:, :], preferred_element_type=jnp.float32))
    out_ref[...] = jnp.maximum(t + bro_ref[...], 0.0)


def _ro(fn_f32, amsg, emb, w_roT, bro2):
    return pl.pallas_call(
        _ro_body,
        grid=(N // _R_TILE,),
        in_specs=[
            pl.BlockSpec((_R_TILE, 1), lambda i: (i, 0)),
            pl.BlockSpec((_R_TILE, D), lambda i: (i, 0)),
            pl.BlockSpec((VOCAB, D), lambda i: (0, 0)),
            pl.BlockSpec((2 * D, D), lambda i: (0, 0)),
            pl.BlockSpec((1, D), lambda i: (0, 0)),
        ],
        out_specs=pl.BlockSpec((_R_TILE, D), lambda i: (i, 0)),
        out_shape=jax.ShapeDtypeStruct((N, D), jnp.float32),
    )(fn_f32, amsg, emb, w_roT, bro2)


def kernel(f_nodes, f_edges, node2edge, edge2node, b2revb, emb, w_edge,
           w_ih, w_hh, b_ih, b_hh, w_ro, b_ro):
    w_edgeT = w_edge.T
    w_ihT = w_ih.T
    w_hhT = w_hh.T
    w_roT = w_ro.T
    bih2 = b_ih.reshape(1, 3 * D)
    bhh2 = b_hh.reshape(1, 3 * D)
    bro2 = b_ro.reshape(1, D)
    n2e_flat = jnp.pad(node2edge.reshape(-1).astype(jnp.int32),
                       (0, (N_IDX_PAD - N) * MAX_NB))
    e2n_pad = jnp.pad(edge2node.astype(jnp.int32), (0, E_IDX_PAD - E))
    brev_pad = jnp.pad(b2revb.astype(jnp.int32), (0, E_IDX_PAD - E))
    fn_f32 = f_nodes.astype(jnp.float32).reshape(N, 1)

    msg = _fe(f_edges, w_edgeT)
    for _ in range(DEPTH - 1):
        amsg = _neisum(n2e_flat, msg)
        mp = _edgemix(e2n_pad, brev_pad, amsg, msg)
        msg = _gru(f_edges, mp, w_edgeT, w_ihT, w_hhT, bih2, bhh2)
    amsg = _neisum(n2e_flat, msg)
    return _ro(fn_f32, amsg, emb, w_roT, bro2)
